# baseline (speedup 1.0000x reference)
import functools

import jax
import jax.numpy as jnp
from jax import lax
from jax.experimental import pallas as pl
from jax.experimental.pallas import tpu as pltpu
from jax.experimental.pallas import tpu_sc as plsc

N = 10000
E = 320000
D = 128
SCALE = 0.1

NC = 2
NS = 16
NW = NC * NS
N_PAD = 10240
ROWS_PER_T = N_PAD // NS
CHUNK = 128
NCHUNK = 80
HALF = NCHUNK // 2
E_PAD = NW * NCHUNK * CHUNK

ROW_BLK = 1000


def _proj_body(f_ref, w_ref, y_ref):
    x = f_ref[...] * SCALE
    w1 = w_ref[:D, :]
    w2 = w_ref[D:, :]
    y_ref[...] = (jnp.dot(x, w1, preferred_element_type=jnp.float32)
                  + jnp.dot(x * x, w2, preferred_element_type=jnp.float32))


_proj = pl.pallas_call(
    _proj_body,
    grid=(N // ROW_BLK,),
    in_specs=[
        pl.BlockSpec((ROW_BLK, D), lambda i: (i, 0)),
        pl.BlockSpec((2 * D, D), lambda i: (0, 0)),
    ],
    out_specs=pl.BlockSpec((ROW_BLK, D), lambda i: (i, 0)),
    out_shape=jax.ShapeDtypeStruct((N, D), jnp.float32),
)


_sc_mesh = plsc.VectorSubcoreMesh(core_axis_name="c", subcore_axis_name="s")


@functools.partial(
    pl.kernel,
    mesh=_sc_mesh,
    out_type=jax.ShapeDtypeStruct((NC, N_PAD, D), jnp.float32),
    scratch_types=[
        pltpu.VMEM((CHUNK,), jnp.int32),
        pltpu.VMEM((CHUNK,), jnp.int32),
        pltpu.VMEM((CHUNK, D), jnp.float32),
        pltpu.VMEM((CHUNK, D), jnp.float32),
        pltpu.VMEM_SHARED((N_PAD, D), jnp.float32),
        pltpu.SemaphoreType.DMA,
        pltpu.SemaphoreType.DMA,
    ],
)
def _sc_scatter(y_hbm, src_hbm, dst_hbm, zeros_hbm, out_hbm,
                sidx, didx, rows0, rows1, acc, sem0, sem1):
    cid = lax.axis_index("c")
    sid = lax.axis_index("s")
    w = cid * NS + sid

    rbase = sid * ROWS_PER_T

    def zbody(k, carry):
        rb = rbase + k * CHUNK
        pltpu.sync_copy(zeros_hbm.at[pl.ds(rb, CHUNK)], rows0)
        pltpu.sync_copy(rows0, acc.at[pl.ds(rb, CHUNK)])
        return carry

    lax.fori_loop(0, ROWS_PER_T // CHUNK, zbody, 0)
    plsc.subcore_barrier()

    cbase = w * NCHUNK

    def pbody(i, carry):
        pltpu.sync_copy(src_hbm.at[cbase + i], sidx)
        pltpu.sync_copy(dst_hbm.at[cbase + i], didx)
        pltpu.async_copy(y_hbm.at[sidx], rows0, sem0).wait()
        pltpu.sync_copy(rows0, acc.at[didx], add=True)
        return carry

    lax.fori_loop(0, NCHUNK, pbody, 0)

    plsc.subcore_barrier()

    def wbody(k, carry):
        rb = rbase + k * CHUNK
        pltpu.sync_copy(acc.at[pl.ds(rb, CHUNK)], rows0)
        pltpu.sync_copy(rows0, out_hbm.at[cid, pl.ds(rb, CHUNK)])
        return carry

    lax.fori_loop(0, ROWS_PER_T // CHUNK, wbody, 0)


def _comb_body(p_ref, y_ref, o_ref):
    o_ref[...] = p_ref[0] + p_ref[1] + y_ref[...]


_comb = pl.pallas_call(
    _comb_body,
    grid=(N // ROW_BLK,),
    in_specs=[
        pl.BlockSpec((NC, ROW_BLK, D), lambda i: (0, i, 0)),
        pl.BlockSpec((ROW_BLK, D), lambda i: (i, 0)),
    ],
    out_specs=pl.BlockSpec((ROW_BLK, D), lambda i: (i, 0)),
    out_shape=jax.ShapeDtypeStruct((N, D), jnp.float32),
)


def kernel(features, edge_index, W):
    pad = E_PAD - E
    src = jnp.concatenate([edge_index[0], jnp.zeros((pad,), jnp.int32)])
    dst = jnp.concatenate([edge_index[1],
                           jnp.full((pad,), N_PAD - 1, jnp.int32)])
    src = src.reshape(NW * NCHUNK, CHUNK)
    dst = dst.reshape(NW * NCHUNK, CHUNK)
    y = _proj(features, W)
    zeros = jnp.zeros((N_PAD, D), jnp.float32)
    parts = _sc_scatter(y, src, dst, zeros)
    return _comb(parts, y)


# speedup vs baseline: 1.7404x; 1.7404x over previous
import functools

import jax
import jax.numpy as jnp
from jax import lax
from jax.experimental import pallas as pl
from jax.experimental.pallas import tpu as pltpu
from jax.experimental.pallas import tpu_sc as plsc

N = 10000
E = 320000
D = 128
SCALE = 0.1

NC = 2
NS = 16
NW = NC * NS
N_PAD = 10240
ROWS_PER_T = N_PAD // NS
CHUNK = 80
NCHUNK = 126
EDGES_PER_W = NCHUNK * CHUNK
E_PAD = NW * EDGES_PER_W

ROW_BLK = 1000


def _proj_body(f_ref, w_ref, y_ref):
    x = f_ref[...] * SCALE
    w1 = w_ref[:D, :]
    w2 = w_ref[D:, :]
    y_ref[...] = (jnp.dot(x, w1, preferred_element_type=jnp.float32)
                  + jnp.dot(x * x, w2, preferred_element_type=jnp.float32))


_proj = pl.pallas_call(
    _proj_body,
    grid=(N // ROW_BLK,),
    in_specs=[
        pl.BlockSpec((ROW_BLK, D), lambda i: (i, 0)),
        pl.BlockSpec((2 * D, D), lambda i: (0, 0)),
    ],
    out_specs=pl.BlockSpec((ROW_BLK, D), lambda i: (i, 0)),
    out_shape=jax.ShapeDtypeStruct((N, D), jnp.float32),
)


_sc_mesh = plsc.VectorSubcoreMesh(core_axis_name="c", subcore_axis_name="s")


@functools.partial(
    pl.kernel,
    mesh=_sc_mesh,
    out_type=jax.ShapeDtypeStruct((NC, N_PAD, D), jnp.float32),
    scratch_types=[
        pltpu.VMEM((CHUNK,), jnp.int32),
        pltpu.VMEM((CHUNK,), jnp.int32),
        pltpu.VMEM((CHUNK,), jnp.int32),
        pltpu.VMEM((CHUNK,), jnp.int32),
        pltpu.VMEM((CHUNK, D), jnp.float32),
        pltpu.VMEM((CHUNK, D), jnp.float32),
        pltpu.VMEM_SHARED((N_PAD, D), jnp.float32),
        pltpu.SemaphoreType.DMA,
        pltpu.SemaphoreType.DMA,
    ],
)
def _sc_scatter(y_hbm, src_hbm, dst_hbm, zeros_hbm, out_hbm,
                sidxA, didxA, sidxB, didxB, rowsA, rowsB, acc, semA, semB):
    cid = lax.axis_index("c")
    sid = lax.axis_index("s")
    w = cid * NS + sid

    rbase = sid * ROWS_PER_T

    def zbody(k, carry):
        rb = rbase + k * CHUNK
        pltpu.sync_copy(zeros_hbm.at[pl.ds(rb, CHUNK)], rowsA)
        pltpu.sync_copy(rowsA, acc.at[pl.ds(rb, CHUNK)])
        return carry

    lax.fori_loop(0, ROWS_PER_T // CHUNK, zbody, 0)
    plsc.subcore_barrier()

    ebase = w * EDGES_PER_W

    def load_idx(c, si, di):
        off = ebase + c * CHUNK
        pltpu.sync_copy(src_hbm.at[pl.ds(off, CHUNK)], si)
        pltpu.sync_copy(dst_hbm.at[pl.ds(off, CHUNK)], di)

    load_idx(0, sidxA, didxA)
    pltpu.async_copy(y_hbm.at[sidxA], rowsA, semA)
    load_idx(1, sidxB, didxB)
    pltpu.async_copy(y_hbm.at[sidxB], rowsB, semB)

    def pbody(jj, carry):
        c = 2 * jj
        pltpu.make_async_copy(y_hbm.at[sidxA], rowsA, semA).wait()
        pltpu.sync_copy(rowsA, acc.at[didxA], add=True)
        load_idx(c + 2, sidxA, didxA)
        pltpu.async_copy(y_hbm.at[sidxA], rowsA, semA)
        pltpu.make_async_copy(y_hbm.at[sidxB], rowsB, semB).wait()
        pltpu.sync_copy(rowsB, acc.at[didxB], add=True)
        load_idx(c + 3, sidxB, didxB)
        pltpu.async_copy(y_hbm.at[sidxB], rowsB, semB)
        return carry

    lax.fori_loop(0, NCHUNK // 2 - 1, pbody, 0)

    pltpu.make_async_copy(y_hbm.at[sidxA], rowsA, semA).wait()
    pltpu.sync_copy(rowsA, acc.at[didxA], add=True)
    pltpu.make_async_copy(y_hbm.at[sidxB], rowsB, semB).wait()
    pltpu.sync_copy(rowsB, acc.at[didxB], add=True)

    plsc.subcore_barrier()

    def wbody(k, carry):
        rb = rbase + k * CHUNK
        pltpu.sync_copy(acc.at[pl.ds(rb, CHUNK)], rowsA)
        pltpu.sync_copy(rowsA, out_hbm.at[cid, pl.ds(rb, CHUNK)])
        return carry

    lax.fori_loop(0, ROWS_PER_T // CHUNK, wbody, 0)


def _comb_body(p_ref, y_ref, o_ref):
    o_ref[...] = p_ref[0] + p_ref[1] + y_ref[...]


_comb = pl.pallas_call(
    _comb_body,
    grid=(N // ROW_BLK,),
    in_specs=[
        pl.BlockSpec((NC, ROW_BLK, D), lambda i: (0, i, 0)),
        pl.BlockSpec((ROW_BLK, D), lambda i: (i, 0)),
    ],
    out_specs=pl.BlockSpec((ROW_BLK, D), lambda i: (i, 0)),
    out_shape=jax.ShapeDtypeStruct((N, D), jnp.float32),
)


def kernel(features, edge_index, W):
    pad = E_PAD - E
    src = jnp.concatenate([edge_index[0], jnp.zeros((pad,), jnp.int32)])
    dst = jnp.concatenate([edge_index[1],
                           jnp.full((pad,), N_PAD - 1, jnp.int32)])
    y = _proj(features, W)
    zeros = jnp.zeros((N_PAD, D), jnp.float32)
    parts = _sc_scatter(y, src, dst, zeros)
    return _comb(parts, y)
